# Initial kernel scaffold; baseline (speedup 1.0000x reference)
import jax
import jax.numpy as jnp
from jax.experimental import pallas as pl


def kernel(x, edge_index, edge_features, num_nodes, Wc, bc, W1, b1, W2, b2, W3, b3, W4, b4):
    raise NotImplementedError("write your pallas kernel here")


# speedup vs baseline: 3.6187x; 3.6187x over previous
import functools

import jax
import jax.numpy as jnp
from jax import lax
from jax.experimental import pallas as pl
from jax.experimental.pallas import tpu as pltpu
from jax.experimental.pallas import tpu_sc as plsc

NC = 2
NS = 16
NW = NC * NS
CH = 80
W = 128


def _mesh():
    return plsc.VectorSubcoreMesh(core_axis_name="c", subcore_axis_name="s")


def _row_part(N):
    rp = -(-N // NS)
    rp = (rp + 7) // 8 * 8
    return rp, NS * rp


def _scatter_h(N, E, D):
    EP = E // NW
    NCH = EP // CH
    RP, NP = _row_part(N)

    @functools.partial(
        pl.kernel,
        mesh=_mesh(),
        out_type=jax.ShapeDtypeStruct((NC, NP, D), jnp.float32),
        scratch_types=[
            pltpu.VMEM((NCH, CH), jnp.int32),
            pltpu.VMEM((NCH, CH), jnp.int32),
            pltpu.VMEM((CH, D), jnp.float32),
            pltpu.VMEM_SHARED((NP, D), jnp.float32),
            pltpu.SemaphoreType.DMA,
        ],
    )
    def k(table, srcr, dstr, zd, out_h, idxs, idxd, rows, acc, sem):
        c = lax.axis_index("c")
        s = lax.axis_index("s")
        wid = s * NC + c
        r0 = s * RP
        pltpu.sync_copy(zd, acc.at[pl.ds(r0, RP)])
        pltpu.sync_copy(srcr.at[wid], idxs)
        pltpu.sync_copy(dstr.at[wid], idxd)
        plsc.subcore_barrier()

        def body(j, carry):
            pltpu.async_copy(table.at[idxs.at[j]], rows, sem).wait()
            pltpu.sync_copy(rows, acc.at[idxd.at[j]], add=True)
            return carry

        lax.fori_loop(0, NCH, body, 0)
        plsc.subcore_barrier()
        pltpu.sync_copy(acc.at[pl.ds(r0, RP)], out_h.at[c, pl.ds(r0, RP)])

    return k


def _scatter_ec(N, E, DE):
    EP = E // NW
    NCH = EP // CH
    RP, NP = _row_part(N)
    FR = CH * DE // W

    @functools.partial(
        pl.kernel,
        mesh=_mesh(),
        out_type=jax.ShapeDtypeStruct((NC, NP, W), jnp.float32),
        scratch_types=[
            pltpu.VMEM((NCH, CH), jnp.int32),
            pltpu.VMEM((FR, W), jnp.float32),
            pltpu.VMEM((CH, W), jnp.float32),
            pltpu.VMEM_SHARED((NP, W), jnp.float32),
        ],
    )
    def k(ef2, dstr, zd, out, idxd, flat, pad, acc):
        c = lax.axis_index("c")
        s = lax.axis_index("s")
        wid = s * NC + c
        r0 = s * RP
        pltpu.sync_copy(zd, acc.at[pl.ds(r0, RP)])
        pltpu.sync_copy(zd.at[pl.ds(0, CH)], pad)
        pltpu.sync_copy(dstr.at[wid], idxd)
        one0 = jnp.where(lax.iota(jnp.int32, 16) == 0, 1.0, 0.0)
        for i in range(CH):
            pad[i, pl.ds(DE, 16)] = one0
        plsc.subcore_barrier()

        def body(j, carry):
            pltpu.sync_copy(ef2.at[wid, j], flat)
            for i in range(CH):
                v = flat[i // (W // DE), pl.ds((i % (W // DE)) * DE, DE)]
                pad[i, pl.ds(0, DE)] = v
            pltpu.sync_copy(pad, acc.at[idxd.at[j]], add=True)
            return carry

        lax.fori_loop(0, NCH, body, 0)
        plsc.subcore_barrier()
        pltpu.sync_copy(acc.at[pl.ds(r0, RP)], out.at[c, pl.ds(r0, RP)])

    return k


def _gather_pairs(N, E, D):
    EP = E // NW
    NCH = EP // CH

    @functools.partial(
        pl.kernel,
        mesh=_mesh(),
        out_type=[
            jax.ShapeDtypeStruct((E, D), jnp.float32),
            jax.ShapeDtypeStruct((E, D), jnp.float32),
        ],
        scratch_types=[
            pltpu.VMEM((NCH, CH), jnp.int32),
            pltpu.VMEM((NCH, CH), jnp.int32),
            pltpu.VMEM((CH, D), jnp.float32),
            pltpu.VMEM((CH, D), jnp.float32),
            pltpu.SemaphoreType.DMA,
            pltpu.SemaphoreType.DMA,
        ],
    )
    def k(h2, srcr, dstr, hv, hu, idxs, idxd, bufa, bufb, sema, semb):
        c = lax.axis_index("c")
        s = lax.axis_index("s")
        wid = s * NC + c
        ebase = wid * EP
        pltpu.sync_copy(srcr.at[wid], idxs)
        pltpu.sync_copy(dstr.at[wid], idxd)

        def body(j, carry):
            pltpu.async_copy(h2.at[idxs.at[j]], bufa, sema).wait()
            pltpu.sync_copy(bufa, hv.at[pl.ds(ebase + j * CH, CH)])
            pltpu.async_copy(h2.at[idxd.at[j]], bufb, semb).wait()
            pltpu.sync_copy(bufb, hu.at[pl.ds(ebase + j * CH, CH)])
            return carry

        lax.fori_loop(0, NCH, body, 0)

    return k


def _node_body(h_ref, sh_ref, sec_ref, wc_ref, bc_ref, o_ref):
    D = h_ref.shape[1]
    DE = wc_ref.shape[0] - 2 * D
    h = h_ref[...]
    shp = sh_ref[...]
    scp = sec_ref[...]
    sh = shp[0] + shp[1]
    sec = scp[0] + scp[1]
    se = sec[:, 0:DE]
    cnt = sec[:, DE:DE + 1]
    inv = 1.0 / jnp.maximum(cnt, 1.0)
    t = jnp.dot(sh, wc_ref[D:2 * D, :], preferred_element_type=jnp.float32)
    t = t + jnp.dot(se, wc_ref[2 * D:, :], preferred_element_type=jnp.float32)
    o = jnp.dot(h, wc_ref[0:D, :], preferred_element_type=jnp.float32)
    o_ref[...] = jnp.maximum(o + t * inv + bc_ref[...], 0.0)


def _node_update(h, sh, sec, Wc, bc2):
    N, D = h.shape
    TN = 2000
    return pl.pallas_call(
        _node_body,
        grid=(N // TN,),
        in_specs=[
            pl.BlockSpec((TN, D), lambda i: (i, 0)),
            pl.BlockSpec((NC, TN, D), lambda i: (0, i, 0)),
            pl.BlockSpec((NC, TN, W), lambda i: (0, i, 0)),
            pl.BlockSpec((2 * D + Wc.shape[0] - 2 * D, D), lambda i: (0, 0)),
            pl.BlockSpec((1, D), lambda i: (0, 0)),
        ],
        out_specs=pl.BlockSpec((TN, D), lambda i: (i, 0)),
        out_shape=jax.ShapeDtypeStruct((N, D), jnp.float32),
    )(h, sh, sec, Wc, bc2)


def _edge_body(hv_ref, hu_ref, ef_ref, w1_ref, b1_ref, w2_ref, b2_ref,
               w3_ref, b3_ref, w4_ref, b4_ref, o_ref):
    D = hv_ref.shape[1]
    z = jnp.dot(hv_ref[...], w1_ref[0:D, :], preferred_element_type=jnp.float32)
    z = z + jnp.dot(hu_ref[...], w1_ref[D:2 * D, :], preferred_element_type=jnp.float32)
    z = z + jnp.dot(ef_ref[...], w1_ref[2 * D:, :], preferred_element_type=jnp.float32)
    z = jnp.maximum(z + b1_ref[...], 0.0)
    z = jnp.maximum(jnp.dot(z, w2_ref[...], preferred_element_type=jnp.float32) + b2_ref[...], 0.0)
    z = jnp.maximum(jnp.dot(z, w3_ref[...], preferred_element_type=jnp.float32) + b3_ref[...], 0.0)
    o_ref[...] = jnp.dot(z, w4_ref[...], preferred_element_type=jnp.float32) + b4_ref[...]


def _edge_mlp(hv, hu, ef, W1, b1, W2, b2, W3, b3, W4, b4):
    E, D = hv.shape
    DE = ef.shape[1]
    H1, H2, H3 = W2.shape[0], W3.shape[0], W4.shape[0]
    TM = 2000
    return pl.pallas_call(
        _edge_body,
        grid=(E // TM,),
        in_specs=[
            pl.BlockSpec((TM, D), lambda i: (i, 0)),
            pl.BlockSpec((TM, D), lambda i: (i, 0)),
            pl.BlockSpec((TM, DE), lambda i: (i, 0)),
            pl.BlockSpec((2 * D + DE, H1), lambda i: (0, 0)),
            pl.BlockSpec((1, H1), lambda i: (0, 0)),
            pl.BlockSpec((H1, H2), lambda i: (0, 0)),
            pl.BlockSpec((1, H2), lambda i: (0, 0)),
            pl.BlockSpec((H2, H3), lambda i: (0, 0)),
            pl.BlockSpec((1, H3), lambda i: (0, 0)),
            pl.BlockSpec((H3, 1), lambda i: (0, 0)),
            pl.BlockSpec((1, 1), lambda i: (0, 0)),
        ],
        out_specs=pl.BlockSpec((TM, 1), lambda i: (i, 0)),
        out_shape=jax.ShapeDtypeStruct((E, 1), jnp.float32),
    )(hv, hu, ef, W1, b1, W2, b2, W3, b3, W4, b4)


def kernel(x, edge_index, edge_features, num_nodes, Wc, bc,
           W1, b1, W2, b2, W3, b3, W4, b4):
    N, D = x.shape
    E = edge_index.shape[1]
    DE = edge_features.shape[1]
    EP = E // NW
    NCH = EP // CH
    RP, _ = _row_part(N)

    src32 = edge_index[0].reshape(NW, NCH, CH)
    dst32 = edge_index[1].reshape(NW, NCH, CH)
    ef2 = edge_features.reshape(NW, NCH, CH * DE // W, W)
    zd = jnp.zeros((RP, D), jnp.float32)
    bc2 = bc.reshape(1, D)
    b1r = b1.reshape(1, -1)
    b2r = b2.reshape(1, -1)
    b3r = b3.reshape(1, -1)
    b4r = b4.reshape(1, -1)

    scat_h = _scatter_h(N, E, D)
    scat_ec = _scatter_ec(N, E, DE)
    gath = _gather_pairs(N, E, D)

    sec = scat_ec(ef2, dst32, zd)
    sh1 = scat_h(x, src32, dst32, zd)
    h1 = _node_update(x, sh1, sec, Wc, bc2)
    sh2 = scat_h(h1, src32, dst32, zd)
    h2 = _node_update(h1, sh2, sec, Wc, bc2)
    hv, hu = gath(h2, src32, dst32)
    return _edge_mlp(hv, hu, edge_features, W1, b1r, W2, b2r, W3, b3r, W4, b4r)


# speedup vs baseline: 4.0892x; 1.1300x over previous
import functools

import jax
import jax.numpy as jnp
from jax import lax
from jax.experimental import pallas as pl
from jax.experimental.pallas import tpu as pltpu
from jax.experimental.pallas import tpu_sc as plsc

NC = 2
NS = 16
NW = NC * NS
CH = 80
W = 128


def _mesh():
    return plsc.VectorSubcoreMesh(core_axis_name="c", subcore_axis_name="s")


def _row_part(N):
    rp = -(-N // NS)
    rp = (rp + 7) // 8 * 8
    return rp, NS * rp


def _scatter_h(N, E, D):
    EP = E // NW
    NCH = EP // CH
    RP, NP = _row_part(N)

    @functools.partial(
        pl.kernel,
        mesh=_mesh(),
        out_type=jax.ShapeDtypeStruct((NC, NP, D), jnp.float32),
        scratch_types=[
            pltpu.VMEM((NCH, CH), jnp.int32),
            pltpu.VMEM((NCH, CH), jnp.int32),
            pltpu.VMEM((CH, D), jnp.float32),
            pltpu.VMEM_SHARED((NP, D), jnp.float32),
            pltpu.SemaphoreType.DMA,
        ],
    )
    def k(table, srcr, dstr, zd, out_h, idxs, idxd, rows, acc, sem):
        c = lax.axis_index("c")
        s = lax.axis_index("s")
        wid = s * NC + c
        r0 = s * RP
        pltpu.sync_copy(zd, acc.at[pl.ds(r0, RP)])
        pltpu.sync_copy(srcr.at[wid], idxs)
        pltpu.sync_copy(dstr.at[wid], idxd)
        plsc.subcore_barrier()

        def body(j, carry):
            pltpu.async_copy(table.at[idxs.at[j]], rows, sem).wait()
            pltpu.sync_copy(rows, acc.at[idxd.at[j]], add=True)
            return carry

        lax.fori_loop(0, NCH, body, 0)
        plsc.subcore_barrier()
        pltpu.sync_copy(acc.at[pl.ds(r0, RP)], out_h.at[c, pl.ds(r0, RP)])

    return k


def _scatter_ec(N, E, DE):
    EP = E // NW
    NCH = EP // CH
    RP, NP = _row_part(N)
    FR = CH * DE // W

    @functools.partial(
        pl.kernel,
        mesh=_mesh(),
        out_type=jax.ShapeDtypeStruct((NC, NP, W), jnp.float32),
        scratch_types=[
            pltpu.VMEM((NCH, CH), jnp.int32),
            pltpu.VMEM((FR, W), jnp.float32),
            pltpu.VMEM((CH, W), jnp.float32),
            pltpu.VMEM_SHARED((NP, W), jnp.float32),
        ],
    )
    def k(ef2, dstr, zd, out, idxd, flat, pad, acc):
        c = lax.axis_index("c")
        s = lax.axis_index("s")
        wid = s * NC + c
        r0 = s * RP
        pltpu.sync_copy(zd, acc.at[pl.ds(r0, RP)])
        pltpu.sync_copy(zd.at[pl.ds(0, CH)], pad)
        pltpu.sync_copy(dstr.at[wid], idxd)
        one0 = jnp.where(lax.iota(jnp.int32, 16) == 0, 1.0, 0.0)
        for i in range(CH):
            pad[i, pl.ds(DE, 16)] = one0
        plsc.subcore_barrier()

        def body(j, carry):
            pltpu.sync_copy(ef2.at[wid, j], flat)
            for i in range(CH):
                v = flat[i // (W // DE), pl.ds((i % (W // DE)) * DE, DE)]
                pad[i, pl.ds(0, DE)] = v
            pltpu.sync_copy(pad, acc.at[idxd.at[j]], add=True)
            return carry

        lax.fori_loop(0, NCH, body, 0)
        plsc.subcore_barrier()
        pltpu.sync_copy(acc.at[pl.ds(r0, RP)], out.at[c, pl.ds(r0, RP)])

    return k


def _gather_pairs(N, E, D):
    EP = E // NW
    NCH = EP // CH

    @functools.partial(
        pl.kernel,
        mesh=_mesh(),
        out_type=[
            jax.ShapeDtypeStruct((E, D), jnp.float32),
            jax.ShapeDtypeStruct((E, D), jnp.float32),
        ],
        scratch_types=[
            pltpu.VMEM((NCH, CH), jnp.int32),
            pltpu.VMEM((NCH, CH), jnp.int32),
            pltpu.VMEM((CH, D), jnp.float32),
            pltpu.VMEM((CH, D), jnp.float32),
            pltpu.SemaphoreType.DMA,
            pltpu.SemaphoreType.DMA,
        ],
    )
    def k(h2, srcr, dstr, hv, hu, idxs, idxd, bufa, bufb, sema, semb):
        c = lax.axis_index("c")
        s = lax.axis_index("s")
        wid = s * NC + c
        ebase = wid * EP
        pltpu.sync_copy(srcr.at[wid], idxs)
        pltpu.sync_copy(dstr.at[wid], idxd)

        def body(j, carry):
            pltpu.async_copy(h2.at[idxs.at[j]], bufa, sema).wait()
            pltpu.sync_copy(bufa, hv.at[pl.ds(ebase + j * CH, CH)])
            pltpu.async_copy(h2.at[idxd.at[j]], bufb, semb).wait()
            pltpu.sync_copy(bufb, hu.at[pl.ds(ebase + j * CH, CH)])
            return carry

        lax.fori_loop(0, NCH, body, 0)

    return k


def _node_body(h_ref, sh_ref, sec_ref, wc_ref, bc_ref, o_ref):
    D = h_ref.shape[1]
    DE = wc_ref.shape[0] - 2 * D
    h = h_ref[...]
    shp = sh_ref[...]
    scp = sec_ref[...]
    sh = shp[0] + shp[1]
    sec = scp[0] + scp[1]
    se = sec[:, 0:DE]
    cnt = sec[:, DE:DE + 1]
    inv = 1.0 / jnp.maximum(cnt, 1.0)
    t = jnp.dot(sh, wc_ref[D:2 * D, :], preferred_element_type=jnp.float32)
    t = t + jnp.dot(se, wc_ref[2 * D:, :], preferred_element_type=jnp.float32)
    o = jnp.dot(h, wc_ref[0:D, :], preferred_element_type=jnp.float32)
    o_ref[...] = jnp.maximum(o + t * inv + bc_ref[...], 0.0)


def _node_update(h, sh, sec, Wc, bc2):
    N, D = h.shape
    TN = 2000
    return pl.pallas_call(
        _node_body,
        grid=(N // TN,),
        in_specs=[
            pl.BlockSpec((TN, D), lambda i: (i, 0)),
            pl.BlockSpec((NC, TN, D), lambda i: (0, i, 0)),
            pl.BlockSpec((NC, TN, W), lambda i: (0, i, 0)),
            pl.BlockSpec((2 * D + Wc.shape[0] - 2 * D, D), lambda i: (0, 0)),
            pl.BlockSpec((1, D), lambda i: (0, 0)),
        ],
        out_specs=pl.BlockSpec((TN, D), lambda i: (i, 0)),
        out_shape=jax.ShapeDtypeStruct((N, D), jnp.float32),
    )(h, sh, sec, Wc, bc2)


def _edge_body(hv_ref, hu_ref, ef_ref, w1_ref, b1_ref, w2_ref, b2_ref,
               w3_ref, b3_ref, w4_ref, b4_ref, o_ref):
    D = hv_ref.shape[1]
    z = jnp.dot(hv_ref[...], w1_ref[0:D, :], preferred_element_type=jnp.float32)
    z = z + jnp.dot(hu_ref[...], w1_ref[D:2 * D, :], preferred_element_type=jnp.float32)
    z = z + jnp.dot(ef_ref[...], w1_ref[2 * D:, :], preferred_element_type=jnp.float32)
    z = jnp.maximum(z + b1_ref[...], 0.0)
    z = jnp.maximum(jnp.dot(z, w2_ref[...], preferred_element_type=jnp.float32) + b2_ref[...], 0.0)
    z = jnp.maximum(jnp.dot(z, w3_ref[...], preferred_element_type=jnp.float32) + b3_ref[...], 0.0)
    o_ref[...] = jnp.dot(z, w4_ref[...], preferred_element_type=jnp.float32) + b4_ref[...]


def _edge_mlp(hv, hu, ef, W1, b1, W2, b2, W3, b3, W4, b4):
    E, D = hv.shape
    DE = ef.shape[1]
    H1, H2, H3 = W2.shape[0], W3.shape[0], W4.shape[0]
    TM = 2000
    return pl.pallas_call(
        _edge_body,
        grid=(E // TM,),
        in_specs=[
            pl.BlockSpec((TM, D), lambda i: (i, 0)),
            pl.BlockSpec((TM, D), lambda i: (i, 0)),
            pl.BlockSpec((TM, DE), lambda i: (i, 0)),
            pl.BlockSpec((2 * D + DE, H1), lambda i: (0, 0)),
            pl.BlockSpec((1, H1), lambda i: (0, 0)),
            pl.BlockSpec((H1, H2), lambda i: (0, 0)),
            pl.BlockSpec((1, H2), lambda i: (0, 0)),
            pl.BlockSpec((H2, H3), lambda i: (0, 0)),
            pl.BlockSpec((1, H3), lambda i: (0, 0)),
            pl.BlockSpec((H3, 1), lambda i: (0, 0)),
            pl.BlockSpec((1, 1), lambda i: (0, 0)),
        ],
        out_specs=pl.BlockSpec((TM, 1), lambda i: (i, 0)),
        out_shape=jax.ShapeDtypeStruct((E, 1), jnp.float32),
    )(hv, hu, ef, W1, b1, W2, b2, W3, b3, W4, b4)


def kernel(x, edge_index, edge_features, num_nodes, Wc, bc,
           W1, b1, W2, b2, W3, b3, W4, b4):
    N, D = x.shape
    E = edge_index.shape[1]
    DE = edge_features.shape[1]
    EP = E // NW
    NCH = EP // CH
    RP, _ = _row_part(N)

    src32 = edge_index[0].reshape(NW, NCH, CH)
    dst32 = edge_index[1].reshape(NW, NCH, CH)
    ef2 = edge_features.reshape(NW, NCH, CH * DE // W, W)
    zd = jnp.zeros((RP, D), jnp.float32)
    bc2 = bc.reshape(1, D)
    b1r = b1.reshape(1, -1)
    b2r = b2.reshape(1, -1)
    b3r = b3.reshape(1, -1)
    b4r = b4.reshape(1, -1)

    scat_h = _scatter_h(N, E, D)
    scat_ec = _scatter_ec(N, E, DE)

    sec = scat_ec(ef2, dst32, zd)
    sh1 = scat_h(x, src32, dst32, zd)
    h1 = _node_update(x, sh1, sec, Wc, bc2)
    sh2 = scat_h(h1, src32, dst32, zd)
    h2 = _node_update(h1, sh2, sec, Wc, bc2)

    KC = 5
    EK = E // KC
    gath = _gather_pairs(N, EK, D)
    srck = edge_index[0].reshape(KC, NW, EK // (NW * CH), CH)
    dstk = edge_index[1].reshape(KC, NW, EK // (NW * CH), CH)
    efk = edge_features.reshape(KC, EK, DE)
    preds = []
    for k in range(KC):
        hv, hu = gath(h2, srck[k], dstk[k])
        preds.append(_edge_mlp(hv, hu, efk[k], W1, b1r, W2, b2r, W3, b3r, W4, b4r))
    return jnp.concatenate(preds, axis=0)


# speedup vs baseline: 4.5293x; 1.1076x over previous
import functools

import jax
import jax.numpy as jnp
from jax import lax
from jax.experimental import pallas as pl
from jax.experimental.pallas import tpu as pltpu
from jax.experimental.pallas import tpu_sc as plsc

NC = 2
NS = 16
NW = NC * NS
CH = 80
W = 128


def _mesh():
    return plsc.VectorSubcoreMesh(core_axis_name="c", subcore_axis_name="s")


def _row_part(N):
    rp = -(-N // NS)
    rp = (rp + 7) // 8 * 8
    return rp, NS * rp


def _scatter_h(N, E, D):
    EP = E // NW
    NCH = EP // CH
    RP, NP = _row_part(N)

    WS = 3
    SB = 5
    BCH = NCH // SB
    WN = BCH // WS
    TL = BCH - WN * WS

    @functools.partial(
        pl.kernel,
        mesh=_mesh(),
        out_type=jax.ShapeDtypeStruct((NC, NP, D), jnp.float32),
        scratch_types=[
            pltpu.VMEM((BCH, CH), jnp.int32),
            pltpu.VMEM((BCH, CH), jnp.int32),
            pltpu.VMEM((WS, CH, D), jnp.float32),
            pltpu.VMEM_SHARED((NP, D), jnp.float32),
            pltpu.SemaphoreType.DMA,
            pltpu.SemaphoreType.DMA,
        ],
    )
    def k(table, srcr, dstr, zd, out_h, idxs, idxd, slots, acc, gsem, ssem):
        c = lax.axis_index("c")
        s = lax.axis_index("s")
        wid = s * NC + c
        r0 = s * RP
        pltpu.sync_copy(zd, acc.at[pl.ds(r0, RP)])
        plsc.subcore_barrier()

        def win(b, n):
            gs = [pltpu.async_copy(table.at[idxs.at[b + i]], slots.at[i], gsem)
                  for i in range(n)]
            sc = []
            for i in range(n):
                gs[i].wait()
                sc.append(pltpu.async_copy(slots.at[i], acc.at[idxd.at[b + i]],
                                           ssem, add=True))
            for h in sc:
                h.wait()

        def wbody(w, carry):
            win(w * WS, WS)
            return carry

        def sblk(t, carry):
            pltpu.sync_copy(srcr.at[wid, t], idxs)
            pltpu.sync_copy(dstr.at[wid, t], idxd)
            lax.fori_loop(0, WN, wbody, 0)
            if TL:
                win(WN * WS, TL)
            return carry

        lax.fori_loop(0, SB, sblk, 0)
        plsc.subcore_barrier()
        pltpu.sync_copy(acc.at[pl.ds(r0, RP)], out_h.at[c, pl.ds(r0, RP)])

    return k


def _scatter_ec(N, E, DE):
    EP = E // NW
    NCH = EP // CH
    RP, NP = _row_part(N)
    FR = CH * DE // W
    WS = 3
    SB = 5
    BCH = NCH // SB
    WN = BCH // WS
    TL = BCH - WN * WS

    @functools.partial(
        pl.kernel,
        mesh=_mesh(),
        out_type=jax.ShapeDtypeStruct((NC, NP, W), jnp.float32),
        scratch_types=[
            pltpu.VMEM((BCH, CH), jnp.int32),
            pltpu.VMEM((WS, FR, W), jnp.float32),
            pltpu.VMEM((WS, CH, W), jnp.float32),
            pltpu.VMEM_SHARED((NP, W), jnp.float32),
            pltpu.SemaphoreType.DMA,
            pltpu.SemaphoreType.DMA,
        ],
    )
    def k(ef2, dstr, zd, out, idxd, flats, pads, acc, lsem, ssem):
        c = lax.axis_index("c")
        s = lax.axis_index("s")
        wid = s * NC + c
        r0 = s * RP
        pltpu.sync_copy(zd, acc.at[pl.ds(r0, RP)])
        one0 = jnp.where(lax.iota(jnp.int32, 16) == 0, 1.0, 0.0)
        for q in range(WS):
            pltpu.sync_copy(zd.at[pl.ds(0, CH)], pads.at[q])
        for q in range(WS):
            for i in range(CH):
                pads[q, i, pl.ds(DE, 16)] = one0
        plsc.subcore_barrier()

        def win(t, b, n):
            lf = [pltpu.async_copy(ef2.at[wid, t, b + q], flats.at[q], lsem)
                  for q in range(n)]
            sc = []
            for q in range(n):
                lf[q].wait()
                for i in range(CH):
                    v = flats[q, i // (W // DE), pl.ds((i % (W // DE)) * DE, DE)]
                    pads[q, i, pl.ds(0, DE)] = v
                sc.append(pltpu.async_copy(pads.at[q], acc.at[idxd.at[b + q]],
                                           ssem, add=True))
            for h in sc:
                h.wait()

        def sblk(t, carry):
            pltpu.sync_copy(dstr.at[wid, t], idxd)

            def wbody(w, cc):
                win(t, w * WS, WS)
                return cc

            lax.fori_loop(0, WN, wbody, 0)
            if TL:
                win(t, WN * WS, TL)
            return carry

        lax.fori_loop(0, SB, sblk, 0)
        plsc.subcore_barrier()
        pltpu.sync_copy(acc.at[pl.ds(r0, RP)], out.at[c, pl.ds(r0, RP)])

    return k


def _gather_pairs(N, E, D):
    EP = E // NW
    NCH = EP // CH

    WS = 5

    @functools.partial(
        pl.kernel,
        mesh=_mesh(),
        out_type=[
            jax.ShapeDtypeStruct((E, D), jnp.float32),
            jax.ShapeDtypeStruct((E, D), jnp.float32),
        ],
        scratch_types=[
            pltpu.VMEM((NCH, CH), jnp.int32),
            pltpu.VMEM((NCH, CH), jnp.int32),
            pltpu.VMEM((WS, CH, D), jnp.float32),
            pltpu.VMEM((WS, CH, D), jnp.float32),
            pltpu.SemaphoreType.DMA,
            pltpu.SemaphoreType.DMA,
            pltpu.SemaphoreType.DMA,
            pltpu.SemaphoreType.DMA,
        ],
    )
    def k(h2, srcr, dstr, hv, hu, idxs, idxd, sa, sb, gsa, gsb, wsa, wsb):
        c = lax.axis_index("c")
        s = lax.axis_index("s")
        wid = s * NC + c
        ebase = wid * EP
        pltpu.sync_copy(srcr.at[wid], idxs)
        pltpu.sync_copy(dstr.at[wid], idxd)

        def body(w, carry):
            b = w * WS
            ga = [pltpu.async_copy(h2.at[idxs.at[b + i]], sa.at[i], gsa)
                  for i in range(WS)]
            gb = [pltpu.async_copy(h2.at[idxd.at[b + i]], sb.at[i], gsb)
                  for i in range(WS)]
            wr = []
            for i in range(WS):
                ga[i].wait()
                wr.append(pltpu.async_copy(
                    sa.at[i], hv.at[pl.ds(ebase + (b + i) * CH, CH)], wsa))
                gb[i].wait()
                wr.append(pltpu.async_copy(
                    sb.at[i], hu.at[pl.ds(ebase + (b + i) * CH, CH)], wsb))
            for h in wr:
                h.wait()
            return carry

        lax.fori_loop(0, NCH // WS, body, 0)

    return k


def _node_body(h_ref, sh_ref, sec_ref, wc_ref, bc_ref, o_ref):
    D = h_ref.shape[1]
    DE = wc_ref.shape[0] - 2 * D
    h = h_ref[...]
    shp = sh_ref[...]
    scp = sec_ref[...]
    sh = shp[0] + shp[1]
    sec = scp[0] + scp[1]
    se = sec[:, 0:DE]
    cnt = sec[:, DE:DE + 1]
    inv = 1.0 / jnp.maximum(cnt, 1.0)
    t = jnp.dot(sh, wc_ref[D:2 * D, :], preferred_element_type=jnp.float32)
    t = t + jnp.dot(se, wc_ref[2 * D:, :], preferred_element_type=jnp.float32)
    o = jnp.dot(h, wc_ref[0:D, :], preferred_element_type=jnp.float32)
    o_ref[...] = jnp.maximum(o + t * inv + bc_ref[...], 0.0)


def _node_update(h, sh, sec, Wc, bc2):
    N, D = h.shape
    TN = 2000
    return pl.pallas_call(
        _node_body,
        grid=(N // TN,),
        in_specs=[
            pl.BlockSpec((TN, D), lambda i: (i, 0)),
            pl.BlockSpec((NC, TN, D), lambda i: (0, i, 0)),
            pl.BlockSpec((NC, TN, W), lambda i: (0, i, 0)),
            pl.BlockSpec((2 * D + Wc.shape[0] - 2 * D, D), lambda i: (0, 0)),
            pl.BlockSpec((1, D), lambda i: (0, 0)),
        ],
        out_specs=pl.BlockSpec((TN, D), lambda i: (i, 0)),
        out_shape=jax.ShapeDtypeStruct((N, D), jnp.float32),
    )(h, sh, sec, Wc, bc2)


def _edge_body(hv_ref, hu_ref, ef_ref, w1_ref, b1_ref, w2_ref, b2_ref,
               w3_ref, b3_ref, w4_ref, b4_ref, o_ref):
    D = hv_ref.shape[1]
    z = jnp.dot(hv_ref[...], w1_ref[0:D, :], preferred_element_type=jnp.float32)
    z = z + jnp.dot(hu_ref[...], w1_ref[D:2 * D, :], preferred_element_type=jnp.float32)
    z = z + jnp.dot(ef_ref[...], w1_ref[2 * D:, :], preferred_element_type=jnp.float32)
    z = jnp.maximum(z + b1_ref[...], 0.0)
    z = jnp.maximum(jnp.dot(z, w2_ref[...], preferred_element_type=jnp.float32) + b2_ref[...], 0.0)
    z = jnp.maximum(jnp.dot(z, w3_ref[...], preferred_element_type=jnp.float32) + b3_ref[...], 0.0)
    o_ref[...] = jnp.dot(z, w4_ref[...], preferred_element_type=jnp.float32) + b4_ref[...]


def _edge_mlp(hv, hu, ef, W1, b1, W2, b2, W3, b3, W4, b4):
    E, D = hv.shape
    DE = ef.shape[1]
    H1, H2, H3 = W2.shape[0], W3.shape[0], W4.shape[0]
    TM = 2000
    return pl.pallas_call(
        _edge_body,
        grid=(E // TM,),
        in_specs=[
            pl.BlockSpec((TM, D), lambda i: (i, 0)),
            pl.BlockSpec((TM, D), lambda i: (i, 0)),
            pl.BlockSpec((TM, DE), lambda i: (i, 0)),
            pl.BlockSpec((2 * D + DE, H1), lambda i: (0, 0)),
            pl.BlockSpec((1, H1), lambda i: (0, 0)),
            pl.BlockSpec((H1, H2), lambda i: (0, 0)),
            pl.BlockSpec((1, H2), lambda i: (0, 0)),
            pl.BlockSpec((H2, H3), lambda i: (0, 0)),
            pl.BlockSpec((1, H3), lambda i: (0, 0)),
            pl.BlockSpec((H3, 1), lambda i: (0, 0)),
            pl.BlockSpec((1, 1), lambda i: (0, 0)),
        ],
        out_specs=pl.BlockSpec((TM, 1), lambda i: (i, 0)),
        out_shape=jax.ShapeDtypeStruct((E, 1), jnp.float32),
    )(hv, hu, ef, W1, b1, W2, b2, W3, b3, W4, b4)


def kernel(x, edge_index, edge_features, num_nodes, Wc, bc,
           W1, b1, W2, b2, W3, b3, W4, b4):
    N, D = x.shape
    E = edge_index.shape[1]
    DE = edge_features.shape[1]
    EP = E // NW
    NCH = EP // CH
    RP, _ = _row_part(N)

    src32 = edge_index[0].reshape(NW, 5, NCH // 5, CH)
    dst32 = edge_index[1].reshape(NW, 5, NCH // 5, CH)
    ef2 = edge_features.reshape(NW, 5, NCH // 5, CH * DE // W, W)
    zd = jnp.zeros((RP, D), jnp.float32)
    bc2 = bc.reshape(1, D)
    b1r = b1.reshape(1, -1)
    b2r = b2.reshape(1, -1)
    b3r = b3.reshape(1, -1)
    b4r = b4.reshape(1, -1)

    scat_h = _scatter_h(N, E, D)
    scat_ec = _scatter_ec(N, E, DE)

    sec = scat_ec(ef2, dst32, zd)
    sh1 = scat_h(x, src32, dst32, zd)
    h1 = _node_update(x, sh1, sec, Wc, bc2)
    sh2 = scat_h(h1, src32, dst32, zd)
    h2 = _node_update(h1, sh2, sec, Wc, bc2)

    KC = 5
    EK = E // KC
    gath = _gather_pairs(N, EK, D)
    srck = edge_index[0].reshape(KC, NW, EK // (NW * CH), CH)
    dstk = edge_index[1].reshape(KC, NW, EK // (NW * CH), CH)
    efk = edge_features.reshape(KC, EK, DE)
    preds = []
    for k in range(KC):
        hv, hu = gath(h2, srck[k], dstk[k])
        preds.append(_edge_mlp(hv, hu, efk[k], W1, b1r, W2, b2r, W3, b3r, W4, b4r))
    return jnp.concatenate(preds, axis=0)
